# baseline (speedup 1.0000x reference)
import jax
import jax.numpy as jnp
from jax import lax
from jax.experimental import pallas as pl
from jax.experimental.pallas import tpu as pltpu
from jax.experimental.pallas import tpu_sc as plsc

_N_NODES = 100000
_N_NODES_PAD = 100352
_STG = 896
_STG_FULL = _N_NODES // _STG
_STG_TAIL = _N_NODES - _STG_FULL * _STG
_N_EDGES = 3200000
_NC = 2
_NS = 16
_NW = _NC * _NS
_E = 1024
_NCHUNK = _N_EDGES // _E
_LAST = _NCHUNK - 1
_ITERS = 98
_G = _E // 16

_HALF_PI = 1.5707963267948966
_C_ATAN = (0.9999998977903125, -0.33331959846433457, 0.19969236562476794,
           -0.1401658988294469, 0.09906106970472799, -0.059367209648048674,
           0.02416624674527195, -0.004668784473913435)


def _rsqrt(x):
    i = plsc.bitcast(x, jnp.int32)
    i = jnp.int32(0x5F3759DF) - lax.shift_right_logical(i, 1)
    y = plsc.bitcast(i, jnp.float32)
    for _ in range(3):
        y = y * (1.5 - 0.5 * x * y * y)
    return y


def _qmul(q, r):
    qx, qy, qz, qw = q
    rx, ry, rz, rw = r
    return [qw * rx + qx * rw + qy * rz - qz * ry,
            qw * ry - qx * rz + qy * rw + qz * rx,
            qw * rz + qx * ry - qy * rx + qz * rw,
            qw * rw - qx * rx - qy * ry - qz * rz]


def _qconj(q):
    return [-q[0], -q[1], -q[2], q[3]]


def _cross(a, b):
    return [a[1] * b[2] - a[2] * b[1],
            a[2] * b[0] - a[0] * b[2],
            a[0] * b[1] - a[1] * b[0]]


def _qrot(q, v):
    qv = q[:3]
    w = q[3]
    t = [2.0 * c for c in _cross(qv, v)]
    ct = _cross(qv, t)
    return [v[i] + w * t[i] + ct[i] for i in range(3)]


def _edge_math(t1, q1, t2, q2, tp, qp):
    qA = _qmul(q2, _qconj(q1))
    rA = _qrot(qA, t1)
    tA = [t2[i] - rA[i] for i in range(3)]
    qip = _qconj(qp)
    qe = _qmul(qip, qA)
    d = [tA[i] - tp[i] for i in range(3)]
    te = _qrot(qip, d)

    w0 = qe[3]
    sflip = jnp.where(w0 < 0.0, jnp.float32(-1.0), jnp.float32(1.0))
    qv = [qe[i] * sflip for i in range(3)]
    w = jnp.abs(w0)
    n2q = qv[0] * qv[0] + qv[1] * qv[1] + qv[2] * qv[2]
    n = n2q * _rsqrt(jnp.maximum(n2q, jnp.float32(1e-30)))
    num = jnp.minimum(n, w)
    den = jnp.maximum(n, w)
    t = num / den
    u = t * t
    a = jnp.float32(_C_ATAN[-1])
    for ck in _C_ATAN[-2::-1]:
        a = a * u + jnp.float32(ck)
    a = a * t
    half = jnp.where(n > w, jnp.float32(_HALF_PI) - a, a)
    theta = 2.0 * half
    eps = jnp.float32(1e-7)
    big_n = n > eps
    w_safe = jnp.where(w > eps, w, jnp.float32(1.0))
    scale = jnp.where(big_n, theta, jnp.float32(2.0)) / jnp.where(big_n, n, w_safe)
    phi = [qv[i] * scale for i in range(3)]
    small = theta < 1e-4
    s = 2.0 * n * w
    c_ = w * w - n2q
    th2 = theta * theta
    denom = 2.0 * theta * s
    denom = jnp.where(jnp.abs(denom) > 1e-12, denom, jnp.float32(1e-12))
    coef = (denom - (1.0 + c_) * th2) / (th2 * denom)
    coef = jnp.where(small, jnp.float32(1.0 / 12.0), coef)
    pxt = _cross(phi, te)
    cpp = _cross(phi, pxt)
    rho = [te[i] - 0.5 * pxt[i] + coef * cpp[i] for i in range(3)]
    return rho + phi


def _sc_body(nodes_hbm, ed_hbm, poses_hbm, tail_hbm, out_hbm,
             tab, stg_in, stg_out, stg_tail, ed, idx0, idx1, nd1, nd2, pv, ov,
             sem_g, sem_ed, sem_p, sem_o):
    c = lax.axis_index("c")
    s = lax.axis_index("s")
    wid = s * _NC + c

    def stage_sub(sub):
        off = sub * _STG
        pltpu.sync_copy(nodes_hbm.at[:, pl.ds(off, _STG)], stg_in)

        @plsc.parallel_loop(0, _STG, step=16, unroll=2)
        def gbody(g0):
            sl = pl.ds(g0, 16)
            for comp in range(7):
                stg_out[comp][sl] = stg_in[comp, sl]
        for comp in range(7):
            pltpu.sync_copy(stg_out[comp], tab[comp].at[pl.ds(off, _STG)])

    for i in range(6):
        stage_sub(s + _NS * i)

    @pl.when(s < _NS - 1)
    def _():
        stage_sub(s + _NS * 6)

    @pl.when(s == _NS - 1)
    def _():
        pltpu.sync_copy(tail_hbm, stg_tail)
        for comp in range(7):
            pltpu.sync_copy(
                stg_tail.at[pl.ds(comp * _STG_TAIL, _STG_TAIL)],
                tab[comp].at[pl.ds(_STG_FULL * _STG, _STG_TAIL)])

    plsc.subcore_barrier()

    def goff_of(k):
        return jnp.minimum(wid + k * _NW, _LAST) * _E

    def issue_ed(k, b):
        pltpu.async_copy(
            ed_hbm.at[:, pl.ds(goff_of(k), _E)], ed[b], sem_ed[b])

    def issue_poses(k, b):
        pltpu.async_copy(
            poses_hbm.at[:, pl.ds(goff_of(k), _E)], pv[b], sem_p[b])

    def build_idx(b):
        @plsc.parallel_loop(0, _E, step=16, unroll=4)
        def jbody(e0):
            sl = pl.ds(e0, 16)
            idx0[b][pl.ds(e0, 16)] = ed[b][0, sl]
            idx0[b][pl.ds(_E + e0, 16)] = ed[b][1, sl]

    def issue_gathers(b):
        for comp in range(7):
            pltpu.async_copy(tab[comp].at[idx0[b]], nd1[b][comp], sem_g[b])

    def wait_gathers(b):
        for comp in range(7):
            pltpu.make_async_copy(
                tab[comp].at[idx0[b]], nd1[b][comp], sem_g[b]).wait()

    def wait_ed(b):
        pltpu.make_async_copy(
            ed_hbm.at[:, pl.ds(0, _E)], ed[b], sem_ed[b]).wait()

    def wait_poses(b):
        pltpu.make_async_copy(
            poses_hbm.at[:, pl.ds(0, _E)], pv[b], sem_p[b]).wait()

    def issue_out(k, b):
        pltpu.async_copy(
            ov[b], out_hbm.at[:, pl.ds(goff_of(k), _E)], sem_o[b])

    def wait_out(b):
        pltpu.make_async_copy(
            ov[b], out_hbm.at[:, pl.ds(0, _E)], sem_o[b]).wait()

    def compute(b):
        @plsc.parallel_loop(0, _E, step=16, unroll=4)
        def group_body(e0):
            sl = pl.ds(e0, 16)
            sl2 = pl.ds(_E + e0, 16)
            t1 = [nd1[b][i][sl] for i in range(3)]
            q1 = [nd1[b][3 + i][sl] for i in range(4)]
            t2 = [nd1[b][i][sl2] for i in range(3)]
            q2 = [nd1[b][3 + i][sl2] for i in range(4)]
            tp = [pv[b][i, sl] for i in range(3)]
            qp = [pv[b][3 + i, sl] for i in range(4)]
            res = _edge_math(t1, q1, t2, q2, tp, qp)
            for i in range(6):
                ov[b][i, sl] = res[i]

    pltpu.sync_copy(ed_hbm.at[:, pl.ds(goff_of(0), _E)], ed[0])
    build_idx(0)
    issue_gathers(0)
    issue_ed(1, 1)
    issue_poses(0, 0)

    def outer(kk, carry):
        for b in (0, 1):
            k = 2 * kk + b
            nb = 1 - b
            wait_ed(nb)
            build_idx(nb)
            issue_gathers(nb)
            issue_ed(k + 2, b)
            issue_poses(k + 1, nb)
            wait_gathers(b)
            wait_poses(b)

            @pl.when(kk > 0)
            def _():
                wait_out(b)

            compute(b)
            issue_out(k, b)
        return carry

    lax.fori_loop(0, _ITERS // 2, outer, 0)

    wait_gathers(0)
    wait_ed(1)
    wait_poses(0)
    wait_out(0)
    wait_out(1)


@jax.jit
def _pose_graph_sc(nodes_t, edges_t, poses_t, tail):
    run = pl.kernel(
        _sc_body,
        out_type=jax.ShapeDtypeStruct((6, _N_EDGES), jnp.float32),
        mesh=plsc.VectorSubcoreMesh(core_axis_name="c", subcore_axis_name="s"),
        compiler_params=pltpu.CompilerParams(needs_layout_passes=False),
        scratch_types=[
            [pltpu.VMEM_SHARED((_N_NODES_PAD,), jnp.float32) for _ in range(7)],
            pltpu.VMEM((7, _STG), jnp.float32),
            [pltpu.VMEM((_STG,), jnp.float32) for _ in range(7)],
            pltpu.VMEM((7 * _STG_TAIL,), jnp.float32),
            [pltpu.VMEM((2, _E), jnp.int32) for _ in range(2)],
            [pltpu.VMEM((2 * _E,), jnp.int32) for _ in range(2)],
            [pltpu.VMEM((_E,), jnp.int32) for _ in range(2)],
            [[pltpu.VMEM((2 * _E,), jnp.float32) for _ in range(7)] for _ in range(2)],
            [[pltpu.VMEM((_E,), jnp.float32) for _ in range(7)] for _ in range(2)],
            [pltpu.VMEM((7, _E), jnp.float32) for _ in range(2)],
            [pltpu.VMEM((6, _E), jnp.float32) for _ in range(2)],
            [pltpu.SemaphoreType.DMA for _ in range(2)],
            [pltpu.SemaphoreType.DMA for _ in range(2)],
            [pltpu.SemaphoreType.DMA for _ in range(2)],
            [pltpu.SemaphoreType.DMA for _ in range(2)],
        ],
    )
    return run(nodes_t, edges_t, poses_t, tail)


def kernel(nodes, edges, poses):
    nodes_t = nodes.T
    tail = nodes_t[:, _STG_FULL * _STG:].reshape(-1)
    out_t = _pose_graph_sc(nodes_t, edges.T, poses.T, tail)
    return out_t.T
